# Initial kernel scaffold; baseline (speedup 1.0000x reference)
import jax
import jax.numpy as jnp
from jax.experimental import pallas as pl


def kernel(x, codebook):
    raise NotImplementedError("write your pallas kernel here")


# speedup vs baseline: 1.6954x; 1.6954x over previous
import jax
import jax.numpy as jnp
from jax import lax
from jax.experimental import pallas as pl
from jax.experimental.pallas import tpu as pltpu

_B, _C, _T = 16, 256, 1024
_K, _E, _G = 1024, 128, 2
_TT = 1024
_P = _T // _TT


def _vq_body(x_ref, cb_ref, cbt_ref, q_ref, idx_ref):
    xb = x_ref[0]
    cb = cb_ref[...]
    e2 = jnp.sum(cb * cb, axis=1)
    x2 = jnp.sum(xb * xb, axis=0)
    xe = lax.dot_general(cb, xb, (((1,), (0,)), ((), ())),
                         preferred_element_type=jnp.float32)
    s = (x2[None, :] + e2[:, None]) - 2.0 * xe
    m = jnp.min(s, axis=0)
    kio = lax.broadcasted_iota(jnp.int32, (_K, _TT), 0)
    idx = jnp.min(jnp.where(s == m[None, :], kio, jnp.int32(_K)), axis=0)
    idx_ref[0, 0, 0] = idx
    oh = (kio == idx[None, :]).astype(jnp.float32)
    q_ref[0, 0] = lax.dot_general(cbt_ref[...], oh, (((1,), (0,)), ((), ())),
                                  precision=lax.Precision.HIGHEST,
                                  preferred_element_type=jnp.float32)


def kernel(x, codebook):
    xin = x.reshape(_B, _E, _G * _T)
    cbt = codebook.T
    grid = (_B, _G, _P)
    q, idx = pl.pallas_call(
        _vq_body,
        grid=grid,
        in_specs=[
            pl.BlockSpec((1, _E, _TT), lambda b, g, p: (b, 0, g * _P + p)),
            pl.BlockSpec((_K, _E), lambda b, g, p: (0, 0)),
            pl.BlockSpec((_E, _K), lambda b, g, p: (0, 0)),
        ],
        out_specs=[
            pl.BlockSpec((1, 1, _E, _TT), lambda b, g, p: (b, g, 0, p)),
            pl.BlockSpec((1, 1, 1, _TT), lambda b, g, p: (b, g, 0, p)),
        ],
        out_shape=[
            jax.ShapeDtypeStruct((_B, _G, _E, _T), jnp.float32),
            jax.ShapeDtypeStruct((_B, _G, 1, _T), jnp.int32),
        ],
        compiler_params=pltpu.CompilerParams(
            dimension_semantics=("parallel", "parallel", "parallel"),
        ),
    )(xin, codebook, cbt)
    quantized = q.reshape(_B, _C, _T)
    indexes = idx.reshape(_B, _G, _T).transpose(1, 0, 2)
    return quantized, indexes


# speedup vs baseline: 2.5207x; 1.4868x over previous
import jax
import jax.numpy as jnp
from jax import lax
from jax.experimental import pallas as pl
from jax.experimental.pallas import tpu as pltpu

_B, _C, _T = 16, 256, 1024
_K, _E, _G = 1024, 128, 2
_TT = 1024
_P = _T // _TT


def _vq_body(x_ref, cb_ref, cbt_hi_ref, cbt_lo_ref, q_ref, idx_ref):
    xb = x_ref[0]
    cb = cb_ref[...]
    e2 = jnp.sum(cb * cb, axis=1)
    x2 = jnp.sum(xb * xb, axis=0)
    xe = lax.dot_general(cb, xb, (((1,), (0,)), ((), ())),
                         preferred_element_type=jnp.float32)
    s = (x2[None, :] + e2[:, None]) - 2.0 * xe
    m = jnp.min(s, axis=0)
    kio = lax.broadcasted_iota(jnp.int32, (_K, _TT), 0)
    idx = jnp.min(jnp.where(s == m[None, :], kio, jnp.int32(_K)), axis=0)
    idx_ref[0, 0, 0] = idx
    oh = (kio == idx[None, :]).astype(jnp.bfloat16)
    dims = (((1,), (0,)), ((), ()))
    q_ref[0, 0] = (
        lax.dot_general(cbt_hi_ref[...], oh, dims,
                        preferred_element_type=jnp.float32)
        + lax.dot_general(cbt_lo_ref[...], oh, dims,
                          preferred_element_type=jnp.float32))


def kernel(x, codebook):
    xin = x.reshape(_B, _E, _G * _T)
    cbt = codebook.T
    cbt_hi = cbt.astype(jnp.bfloat16)
    cbt_lo = (cbt - lax.optimization_barrier(cbt_hi).astype(jnp.float32)
              ).astype(jnp.bfloat16)
    grid = (_B, _G, _P)
    q, idx = pl.pallas_call(
        _vq_body,
        grid=grid,
        in_specs=[
            pl.BlockSpec((1, _E, _TT), lambda b, g, p: (b, 0, g * _P + p)),
            pl.BlockSpec((_K, _E), lambda b, g, p: (0, 0)),
            pl.BlockSpec((_E, _K), lambda b, g, p: (0, 0)),
            pl.BlockSpec((_E, _K), lambda b, g, p: (0, 0)),
        ],
        out_specs=[
            pl.BlockSpec((1, 1, _E, _TT), lambda b, g, p: (b, g, 0, p)),
            pl.BlockSpec((1, 1, 1, _TT), lambda b, g, p: (b, g, 0, p)),
        ],
        out_shape=[
            jax.ShapeDtypeStruct((_B, _G, _E, _T), jnp.float32),
            jax.ShapeDtypeStruct((_B, _G, 1, _T), jnp.int32),
        ],
        compiler_params=pltpu.CompilerParams(
            dimension_semantics=("parallel", "parallel", "parallel"),
        ),
    )(xin, codebook, cbt_hi, cbt_lo)
    quantized = q.reshape(_B, _C, _T)
    indexes = idx.reshape(_B, _G, _T).transpose(1, 0, 2)
    return quantized, indexes
